# baseline (speedup 1.0000x reference)
import jax
import jax.numpy as jnp
from jax.experimental import pallas as pl
from jax.experimental.pallas import tpu as pltpu

_B, _M, _N, _D = 16, 8192, 8192, 256
_PR = 8
_BB = 1
_BM = 8192


def _proj_body(idx_ref, ishape_ref, pts_ref, wt_ref, g_ref):
    b = pl.program_id(0)

    hf = ishape_ref[2].astype(jnp.float32)
    wf = ishape_ref[3].astype(jnp.float32)
    kp_scale = jnp.maximum(wf, hf) * 0.7
    max_len = jnp.sqrt(wf * wf + hf * hf)
    len_scale = max_len * 0.7

    r = idx_ref[b] % _PR
    x = pts_ref[0, r, 0]
    y = pts_ref[0, r, 1]
    ln = pts_ref[0, r, 3]
    an = pts_ref[0, r, 4]

    nx = (x - wf * 0.5) / kp_scale
    ny = (y - hf * 0.5) / kp_scale
    na = (an - 45.0) / (180.0 * 0.7)
    nl = (ln - len_scale * 0.5) / len_scale

    g_ref[0, 0, :] = (nx * wt_ref[0, :] + ny * wt_ref[1, :]
                      + na * wt_ref[2, :] + nl * wt_ref[3, :])


def _stream_body(num_ref, g_ref, emb_ref, out_ref):
    b = pl.program_id(0)
    row = jax.lax.broadcasted_iota(jnp.int32, (_BB, _BM, 1), 1)
    nums = jnp.stack([num_ref[_BB * b + s] for s in range(_BB)])
    mask = row < nums.reshape(_BB, 1, 1)
    out_ref[...] = emb_ref[...] + jnp.where(mask, g_ref[...], 0.0)


@jax.jit
def kernel(emb, num, pts, indeces, image_shape, W):
    num = num.astype(jnp.int32)
    indeces = indeces.astype(jnp.int32)
    image_shape = image_shape.astype(jnp.int32)
    wt = W.T

    g = pl.pallas_call(
        _proj_body,
        grid_spec=pltpu.PrefetchScalarGridSpec(
            num_scalar_prefetch=2,
            grid=(_B,),
            in_specs=[
                pl.BlockSpec((1, _PR, 5), lambda b, idx, s: (b, idx[b] // _PR, 0)),
                pl.BlockSpec((4, _D), lambda b, idx, s: (0, 0)),
            ],
            out_specs=pl.BlockSpec((1, 1, _D), lambda b, idx, s: (b, 0, 0)),
        ),
        out_shape=jax.ShapeDtypeStruct((_B, 1, _D), jnp.float32),
    )(indeces, image_shape, pts, wt)

    return pl.pallas_call(
        _stream_body,
        grid_spec=pltpu.PrefetchScalarGridSpec(
            num_scalar_prefetch=1,
            grid=(_B // _BB,),
            in_specs=[
                pl.BlockSpec((_BB, 1, _D), lambda b, n: (b, 0, 0)),
                pl.BlockSpec((_BB, _BM, _D), lambda b, n: (b, 0, 0)),
            ],
            out_specs=pl.BlockSpec((_BB, _BM, _D), lambda b, n: (b, 0, 0)),
        ),
        out_shape=jax.ShapeDtypeStruct((_B, _M, _D), emb.dtype),
        compiler_params=pltpu.CompilerParams(
            dimension_semantics=("parallel",),
        ),
    )(num, g, emb)
